# Initial kernel scaffold; baseline (speedup 1.0000x reference)
import jax
import jax.numpy as jnp
from jax.experimental import pallas as pl


def kernel(x, edge_index, W1, b1, W2, b2, Wh, bh):
    raise NotImplementedError("write your pallas kernel here")


# speedup vs baseline: 21.9459x; 21.9459x over previous
import functools

import jax
import jax.numpy as jnp
from jax import lax
from jax.experimental import pallas as pl
from jax.experimental.pallas import tpu as pltpu
from jax.experimental.pallas import tpu_sc as plsc

N = 10000
E = 320000
D = 128

NC = 2
NS = 16
NW = NC * NS

K = 64
C = 160
EPW = C * K
E_PAD = NW * EPW
N_PAD = 10240
RPT = N_PAD // NS
NB = 2

_f32 = jnp.float32
_mesh = plsc.VectorSubcoreMesh(
    core_axis_name="c", subcore_axis_name="s", num_cores=NC, num_subcores=NS
)


def _make_deg_kernel():
  @functools.partial(
      pl.kernel,
      out_type=jax.ShapeDtypeStruct((NC, N_PAD, 16), _f32),
      mesh=_mesh,
      scratch_types=[
          pltpu.VMEM((K,), jnp.int32),
          pltpu.VMEM((K,), jnp.int32),
          pltpu.VMEM((K, 16), _f32),
          pltpu.VMEM((K, 16), _f32),
          pltpu.VMEM_SHARED((N_PAD, 16), _f32),
          pltpu.SemaphoreType.DMA,
          pltpu.SemaphoreType.DMA,
          pltpu.SemaphoreType.DMA,
          pltpu.SemaphoreType.DMA,
      ],
  )
  def deg_kernel(rows_h, out_h, ri0, ri1, ones_v, zero_v, deg_sh,
                 is0, is1, ss0, ss1):
    cid = lax.axis_index("c")
    sid = lax.axis_index("s")
    wid = sid * NC + cid
    ris = (ri0, ri1)
    isems = (is0, is1)
    ssems = (ss0, ss1)

    ones16 = jnp.ones((16,), _f32)
    zeros16 = jnp.zeros((16,), _f32)

    @pl.loop(0, K)
    def _(i):
      ones_v[i, :] = ones16
      zero_v[i, :] = zeros16

    base = sid * RPT
    for t in range(RPT // K):
      pltpu.sync_copy(zero_v, deg_sh.at[pl.ds(base + t * K, K)])
    plsc.subcore_barrier()

    def idx_load(cur, b):
      pltpu.async_copy(rows_h.at[wid, cur], ris[b], isems[b])

    def step(cur, b, more):
      pltpu.make_async_copy(rows_h.at[wid, cur], ris[b], isems[b]).wait()
      pltpu.async_copy(ones_v, deg_sh.at[ris[b]], ssems[b], add=True)
      pltpu.make_async_copy(ones_v, deg_sh.at[ris[b]], ssems[b]).wait()
      if more:
        idx_load(cur + NB, b)

    idx_load(0, 0)
    idx_load(1, 1)

    @pl.loop(0, C - NB, step=NB)
    def _(j):
      for b in range(NB):
        step(j + b, b, True)

    for b in range(NB):
      step(C - NB + b, b, False)

    plsc.subcore_barrier()
    pltpu.sync_copy(
        deg_sh.at[pl.ds(base, RPT)], out_h.at[cid, pl.ds(base, RPT)]
    )

  return deg_kernel


def _make_agg_kernel():
  @functools.partial(
      pl.kernel,
      out_type=jax.ShapeDtypeStruct((NC, N_PAD, D), _f32),
      mesh=_mesh,
      scratch_types=[
          pltpu.VMEM((K,), jnp.int32),
          pltpu.VMEM((K,), jnp.int32),
          pltpu.VMEM((K,), jnp.int32),
          pltpu.VMEM((K,), jnp.int32),
          pltpu.VMEM((K, D), _f32),
          pltpu.VMEM((K, D), _f32),
          pltpu.VMEM_SHARED((N_PAD, D), _f32),
          pltpu.SemaphoreType.DMA,
          pltpu.SemaphoreType.DMA,
          pltpu.SemaphoreType.DMA,
          pltpu.SemaphoreType.DMA,
          pltpu.SemaphoreType.DMA,
          pltpu.SemaphoreType.DMA,
      ],
  )
  def agg_kernel(
      hp_h, rows_h, cols_h, out_h,
      ri0, ri1, ci0, ci1, buf0, buf1, agg_sh,
      is0, is1, gs0, gs1, ss0, ss1,
  ):
    cid = lax.axis_index("c")
    sid = lax.axis_index("s")
    wid = sid * NC + cid
    ris = (ri0, ri1)
    cis = (ci0, ci1)
    bufs = (buf0, buf1)
    isems = (is0, is1)
    gsems = (gs0, gs1)
    ssems = (ss0, ss1)

    zeros16 = jnp.zeros((16,), _f32)

    @pl.loop(0, K)
    def _(i):
      for j in range(D // 16):
        buf0[i, pl.ds(j * 16, 16)] = zeros16

    base = sid * RPT
    for t in range(RPT // K):
      pltpu.sync_copy(buf0, agg_sh.at[pl.ds(base + t * K, K)])
    plsc.subcore_barrier()

    def idx_load(cur, b):
      pltpu.async_copy(rows_h.at[wid, cur], ris[b], isems[b])
      pltpu.async_copy(cols_h.at[wid, cur], cis[b], isems[b])

    def idx_wait(cur, b):
      pltpu.make_async_copy(rows_h.at[wid, cur], ris[b], isems[b]).wait()
      pltpu.make_async_copy(cols_h.at[wid, cur], cis[b], isems[b]).wait()

    def gather_start(b):
      pltpu.async_copy(hp_h.at[ris[b]], bufs[b], gsems[b])

    def gather_wait(b):
      pltpu.make_async_copy(hp_h.at[ris[b]], bufs[b], gsems[b]).wait()

    def step(cur, b, has_next, has_next2):
      if has_next:
        idx_wait(cur + 1, 1 - b)
        gather_start(1 - b)
      gather_wait(b)
      pltpu.async_copy(bufs[b], agg_sh.at[cis[b]], ssems[b], add=True)
      pltpu.make_async_copy(bufs[b], agg_sh.at[cis[b]], ssems[b]).wait()
      if has_next2:
        idx_load(cur + NB, b)

    idx_load(0, 0)
    idx_load(1, 1)
    idx_wait(0, 0)
    gather_start(0)

    @pl.loop(0, C - NB, step=NB)
    def _(j):
      for b in range(NB):
        step(j + b, b, True, True)

    step(C - NB, 0, True, False)
    step(C - 1, 1, False, False)

    plsc.subcore_barrier()
    pltpu.sync_copy(
        agg_sh.at[pl.ds(base, RPT)], out_h.at[cid, pl.ds(base, RPT)]
    )

  return agg_kernel


_BLK = 2000


def _g_of(deg_ref):
  d = deg_ref[0, :, 0:1] + deg_ref[1, :, 0:1] + 1.0
  return lax.rsqrt(d)


def _ka_body(x_ref, w_ref, b_ref, deg_ref, o_ref):
  g = _g_of(deg_ref)
  h = jnp.dot(x_ref[...], w_ref[...], preferred_element_type=_f32) + b_ref[...]
  o_ref[...] = g * h


def _kb_body(agg_ref, hp_ref, deg_ref, w_ref, b_ref, o_ref):
  g = _g_of(deg_ref)
  x2 = jnp.maximum(g * (agg_ref[0] + agg_ref[1] + hp_ref[...]), 0.0)
  h = jnp.dot(x2, w_ref[...], preferred_element_type=_f32) + b_ref[...]
  o_ref[...] = g * h


def _kc_body(agg_ref, hp_ref, deg_ref, w_ref, b_ref, o_ref):
  g = _g_of(deg_ref)
  x3 = jnp.maximum(g * (agg_ref[0] + agg_ref[1] + hp_ref[...]), 0.0)
  o_ref[...] = (
      jnp.dot(x3, w_ref[...], preferred_element_type=_f32) + b_ref[...]
  )


def _tc_call(body, n_stacked_inputs):
  grid = (N // _BLK,)
  stacked_spec16 = pl.BlockSpec((NC, _BLK, 16), lambda i: (0, i, 0))
  stacked_specD = pl.BlockSpec((NC, _BLK, D), lambda i: (0, i, 0))
  rows_spec = pl.BlockSpec((_BLK, D), lambda i: (i, 0))
  w_spec = pl.BlockSpec((D, D), lambda i: (0, 0))
  b_spec = pl.BlockSpec((1, D), lambda i: (0, 0))
  if n_stacked_inputs == 0:
    in_specs = [rows_spec, w_spec, b_spec, stacked_spec16]
  else:
    in_specs = [stacked_specD, rows_spec, stacked_spec16, w_spec, b_spec]
  return pl.pallas_call(
      body,
      grid=grid,
      in_specs=in_specs,
      out_specs=rows_spec,
      out_shape=jax.ShapeDtypeStruct((N, D), _f32),
  )


_deg_kernel = _make_deg_kernel()
_agg_kernel = _make_agg_kernel()
_ka = _tc_call(_ka_body, 0)
_kb = _tc_call(_kb_body, 1)
_kc = _tc_call(_kc_body, 1)


@jax.jit
def kernel(x, edge_index, W1, b1, W2, b2, Wh, bh):
  pad = E_PAD - E
  pad_i = jnp.arange(pad, dtype=jnp.int32)
  rows_gather = jnp.concatenate([edge_index[0], pad_i % N]).reshape(NW, C, K)
  dummy = N + (pad_i % (N_PAD - N))
  rows_deg = jnp.concatenate([edge_index[0], dummy]).reshape(NW, C, K)
  cols_scatter = jnp.concatenate([edge_index[1], dummy]).reshape(NW, C, K)

  deg = _deg_kernel(rows_deg)
  h1p = _ka(x, W1, b1.reshape(1, D), deg)
  agg1 = _agg_kernel(h1p, rows_gather, cols_scatter)
  h2p = _kb(agg1, h1p, deg, W2, b2.reshape(1, D))
  agg2 = _agg_kernel(h2p, rows_gather, cols_scatter)
  return _kc(agg2, h2p, deg, Wh, bh.reshape(1, D))


# speedup vs baseline: 29.2770x; 1.3341x over previous
import functools

import jax
import jax.numpy as jnp
from jax import lax
from jax.experimental import pallas as pl
from jax.experimental.pallas import tpu as pltpu
from jax.experimental.pallas import tpu_sc as plsc

N = 10000
E = 320000
D = 128

NC = 2
NS = 16
NW = NC * NS

K = 128
C = 80
EPW = C * K
E_PAD = NW * EPW
N_PAD = 10240
RPT = N_PAD // NS
NB = 2

_f32 = jnp.float32
_mesh = plsc.VectorSubcoreMesh(
    core_axis_name="c", subcore_axis_name="s", num_cores=NC, num_subcores=NS
)


def _make_deg_kernel():
  @functools.partial(
      pl.kernel,
      out_type=jax.ShapeDtypeStruct((NC, N_PAD, 16), _f32),
      mesh=_mesh,
      scratch_types=[
          pltpu.VMEM((K,), jnp.int32),
          pltpu.VMEM((K,), jnp.int32),
          pltpu.VMEM((K, 16), _f32),
          pltpu.VMEM((K, 16), _f32),
          pltpu.VMEM_SHARED((N_PAD, 16), _f32),
          pltpu.SemaphoreType.DMA,
          pltpu.SemaphoreType.DMA,
          pltpu.SemaphoreType.DMA,
          pltpu.SemaphoreType.DMA,
      ],
  )
  def deg_kernel(rows_h, out_h, ri0, ri1, ones_v, zero_v, deg_sh,
                 is0, is1, ss0, ss1):
    cid = lax.axis_index("c")
    sid = lax.axis_index("s")
    wid = sid * NC + cid
    ris = (ri0, ri1)
    isems = (is0, is1)
    ssems = (ss0, ss1)

    ones16 = jnp.ones((16,), _f32)
    zeros16 = jnp.zeros((16,), _f32)

    @pl.loop(0, K)
    def _(i):
      ones_v[i, :] = ones16
      zero_v[i, :] = zeros16

    base = sid * RPT
    for t in range(RPT // K):
      pltpu.sync_copy(zero_v, deg_sh.at[pl.ds(base + t * K, K)])
    plsc.subcore_barrier()

    def idx_load(cur, b):
      pltpu.async_copy(rows_h.at[wid, cur], ris[b], isems[b])

    def step(cur, b, more):
      pltpu.make_async_copy(rows_h.at[wid, cur], ris[b], isems[b]).wait()
      pltpu.async_copy(ones_v, deg_sh.at[ris[b]], ssems[b], add=True)
      pltpu.make_async_copy(ones_v, deg_sh.at[ris[b]], ssems[b]).wait()
      if more:
        idx_load(cur + NB, b)

    idx_load(0, 0)
    idx_load(1, 1)

    @pl.loop(0, C - NB, step=NB)
    def _(j):
      for b in range(NB):
        step(j + b, b, True)

    for b in range(NB):
      step(C - NB + b, b, False)

    plsc.subcore_barrier()
    pltpu.sync_copy(
        deg_sh.at[pl.ds(base, RPT)], out_h.at[cid, pl.ds(base, RPT)]
    )

  return deg_kernel


def _make_agg_kernel():
  @functools.partial(
      pl.kernel,
      out_type=jax.ShapeDtypeStruct((NC, N_PAD, D), _f32),
      mesh=_mesh,
      scratch_types=[
          pltpu.VMEM((K,), jnp.int32),
          pltpu.VMEM((K,), jnp.int32),
          pltpu.VMEM((K,), jnp.int32),
          pltpu.VMEM((K,), jnp.int32),
          pltpu.VMEM((K, D), _f32),
          pltpu.VMEM((K, D), _f32),
          pltpu.VMEM_SHARED((N_PAD, D), _f32),
          pltpu.SemaphoreType.DMA,
          pltpu.SemaphoreType.DMA,
          pltpu.SemaphoreType.DMA,
          pltpu.SemaphoreType.DMA,
          pltpu.SemaphoreType.DMA,
          pltpu.SemaphoreType.DMA,
      ],
  )
  def agg_kernel(
      hp_h, rows_h, cols_h, out_h,
      ri0, ri1, ci0, ci1, buf0, buf1, agg_sh,
      is0, is1, gs0, gs1, ss0, ss1,
  ):
    cid = lax.axis_index("c")
    sid = lax.axis_index("s")
    wid = sid * NC + cid
    ris = (ri0, ri1)
    cis = (ci0, ci1)
    bufs = (buf0, buf1)
    isems = (is0, is1)
    gsems = (gs0, gs1)
    ssems = (ss0, ss1)

    zeros16 = jnp.zeros((16,), _f32)

    @pl.loop(0, K)
    def _(i):
      for j in range(D // 16):
        buf0[i, pl.ds(j * 16, 16)] = zeros16

    base = sid * RPT
    for t in range(RPT // K):
      pltpu.sync_copy(buf0, agg_sh.at[pl.ds(base + t * K, K)])
    plsc.subcore_barrier()

    def idx_load(cur, b):
      pltpu.async_copy(rows_h.at[wid, cur], ris[b], isems[b])
      pltpu.async_copy(cols_h.at[wid, cur], cis[b], isems[b])

    def idx_wait(cur, b):
      pltpu.make_async_copy(rows_h.at[wid, cur], ris[b], isems[b]).wait()
      pltpu.make_async_copy(cols_h.at[wid, cur], cis[b], isems[b]).wait()

    def gather_start(b):
      pltpu.async_copy(hp_h.at[ris[b]], bufs[b], gsems[b])

    def gather_wait(b):
      pltpu.make_async_copy(hp_h.at[ris[b]], bufs[b], gsems[b]).wait()

    def step(cur, b, has_next, has_next2):
      if has_next:
        idx_wait(cur + 1, 1 - b)
        gather_start(1 - b)
      gather_wait(b)
      pltpu.async_copy(bufs[b], agg_sh.at[cis[b]], ssems[b], add=True)
      pltpu.make_async_copy(bufs[b], agg_sh.at[cis[b]], ssems[b]).wait()
      if has_next2:
        idx_load(cur + NB, b)

    idx_load(0, 0)
    idx_load(1, 1)
    idx_wait(0, 0)
    gather_start(0)

    @pl.loop(0, C - NB, step=NB)
    def _(j):
      for b in range(NB):
        step(j + b, b, True, True)

    step(C - NB, 0, True, False)
    step(C - 1, 1, False, False)

    plsc.subcore_barrier()
    pltpu.sync_copy(
        agg_sh.at[pl.ds(base, RPT)], out_h.at[cid, pl.ds(base, RPT)]
    )

  return agg_kernel


_BLK = 2000


def _g_of(deg_ref):
  d = deg_ref[0, :, 0:1] + deg_ref[1, :, 0:1] + 1.0
  return lax.rsqrt(d)


def _ka_body(x_ref, w_ref, b_ref, deg_ref, o_ref):
  g = _g_of(deg_ref)
  h = jnp.dot(x_ref[...], w_ref[...], preferred_element_type=_f32) + b_ref[...]
  o_ref[...] = g * h


def _kb_body(agg_ref, hp_ref, deg_ref, w_ref, b_ref, o_ref):
  g = _g_of(deg_ref)
  x2 = jnp.maximum(g * (agg_ref[0] + agg_ref[1] + hp_ref[...]), 0.0)
  h = jnp.dot(x2, w_ref[...], preferred_element_type=_f32) + b_ref[...]
  o_ref[...] = g * h


def _kc_body(agg_ref, hp_ref, deg_ref, w_ref, b_ref, o_ref):
  g = _g_of(deg_ref)
  x3 = jnp.maximum(g * (agg_ref[0] + agg_ref[1] + hp_ref[...]), 0.0)
  o_ref[...] = (
      jnp.dot(x3, w_ref[...], preferred_element_type=_f32) + b_ref[...]
  )


def _tc_call(body, n_stacked_inputs):
  grid = (N // _BLK,)
  stacked_spec16 = pl.BlockSpec((NC, _BLK, 16), lambda i: (0, i, 0))
  stacked_specD = pl.BlockSpec((NC, _BLK, D), lambda i: (0, i, 0))
  rows_spec = pl.BlockSpec((_BLK, D), lambda i: (i, 0))
  w_spec = pl.BlockSpec((D, D), lambda i: (0, 0))
  b_spec = pl.BlockSpec((1, D), lambda i: (0, 0))
  if n_stacked_inputs == 0:
    in_specs = [rows_spec, w_spec, b_spec, stacked_spec16]
  else:
    in_specs = [stacked_specD, rows_spec, stacked_spec16, w_spec, b_spec]
  return pl.pallas_call(
      body,
      grid=grid,
      in_specs=in_specs,
      out_specs=rows_spec,
      out_shape=jax.ShapeDtypeStruct((N, D), _f32),
  )


_deg_kernel = _make_deg_kernel()
_agg_kernel = _make_agg_kernel()
_ka = _tc_call(_ka_body, 0)
_kb = _tc_call(_kb_body, 1)
_kc = _tc_call(_kc_body, 1)


@jax.jit
def kernel(x, edge_index, W1, b1, W2, b2, Wh, bh):
  pad = E_PAD - E
  pad_i = jnp.arange(pad, dtype=jnp.int32)
  rows_gather = jnp.concatenate([edge_index[0], pad_i % N]).reshape(NW, C, K)
  dummy = N + (pad_i % (N_PAD - N))
  rows_deg = jnp.concatenate([edge_index[0], dummy]).reshape(NW, C, K)
  cols_scatter = jnp.concatenate([edge_index[1], dummy]).reshape(NW, C, K)

  deg = _deg_kernel(rows_deg)
  h1p = _ka(x, W1, b1.reshape(1, D), deg)
  agg1 = _agg_kernel(h1p, rows_gather, cols_scatter)
  h2p = _kb(agg1, h1p, deg, W2, b2.reshape(1, D))
  agg2 = _agg_kernel(h2p, rows_gather, cols_scatter)
  return _kc(agg2, h2p, deg, Wh, bh.reshape(1, D))


# speedup vs baseline: 33.1883x; 1.1336x over previous
import functools

import jax
import jax.numpy as jnp
from jax import lax
from jax.experimental import pallas as pl
from jax.experimental.pallas import tpu as pltpu
from jax.experimental.pallas import tpu_sc as plsc

N = 10000
E = 320000
D = 128

NC = 2
NS = 16
NW = NC * NS

K = 128
C = 80
EPW = C * K
E_PAD = NW * EPW
N_TAB = 10240
RPT = 640
RPT_LAST = N_TAB - 15 * RPT
DB = 2
NI = 5

import math
_U = math.lcm(DB, NI)
_LOOP_END = DB + ((C - 2 - DB) // _U) * _U

_f32 = jnp.float32
_mesh = plsc.VectorSubcoreMesh(
    core_axis_name="c", subcore_axis_name="s", num_cores=NC, num_subcores=NS
)


def _table_io(sh, out_or_none, sid, cid, zero_src=None):
  base = sid * RPT

  def copies(n_rows):
    for t in range(n_rows // K):
      if zero_src is not None:
        pltpu.sync_copy(zero_src, sh.at[pl.ds(base + t * K, K)])
      else:
        pltpu.sync_copy(sh.at[pl.ds(base + t * K, K)],
                        out_or_none.at[cid, pl.ds(base + t * K, K)])
    rem, off = n_rows % K, (n_rows // K) * K
    if rem:
      if zero_src is not None:
        pltpu.sync_copy(zero_src.at[pl.ds(0, rem)],
                        sh.at[pl.ds(base + off, rem)])
      else:
        pltpu.sync_copy(sh.at[pl.ds(base + off, rem)],
                        out_or_none.at[cid, pl.ds(base + off, rem)])

  if RPT_LAST == RPT:
    copies(RPT)
  else:
    @pl.when(sid < NS - 1)
    def _():
      copies(RPT)

    @pl.when(sid == NS - 1)
    def _():
      copies(RPT_LAST)


def _make_deg_kernel():
  @functools.partial(
      pl.kernel,
      out_type=jax.ShapeDtypeStruct((NC, N_TAB, 16), _f32),
      mesh=_mesh,
      scratch_types=(
          [pltpu.VMEM((K,), jnp.int32) for _ in range(NI)]
          + [
              pltpu.VMEM((K, 16), _f32),
              pltpu.VMEM((K, 16), _f32),
              pltpu.VMEM_SHARED((N_TAB, 16), _f32),
          ]
          + [pltpu.SemaphoreType.DMA] * (NI + DB)
      ),
  )
  def deg_kernel(rows_h, out_h, *refs):
    ris = refs[0:NI]
    ones_v, zero_v, deg_sh = refs[NI:NI + 3]
    isems = refs[NI + 3:2 * NI + 3]
    ssems = refs[2 * NI + 3:2 * NI + 3 + DB]
    cid = lax.axis_index("c")
    sid = lax.axis_index("s")
    wid = sid * NC + cid

    ones16 = jnp.ones((16,), _f32)
    zeros16 = jnp.zeros((16,), _f32)

    @pl.loop(0, K)
    def _(i):
      ones_v[i, :] = ones16
      zero_v[i, :] = zeros16

    _table_io(deg_sh, None, sid, cid, zero_src=zero_v)
    plsc.subcore_barrier()

    def idx_load(cur, i):
      pltpu.async_copy(rows_h.at[wid, cur], ris[i], isems[i])

    def idx_wait(cur, i):
      pltpu.make_async_copy(rows_h.at[wid, cur], ris[i], isems[i]).wait()

    def s_start(i, b):
      pltpu.async_copy(ones_v, deg_sh.at[ris[i]], ssems[b], add=True)

    def s_wait(i, b):
      pltpu.make_async_copy(ones_v, deg_sh.at[ris[i]], ssems[b]).wait()

    def step(cur, i, b, load):
      if load:
        idx_load(cur + 2, (i + 2) % NI)
      idx_wait(cur, i)
      s_start(i, b)
      s_wait(i, b)

    idx_load(0, 0)
    idx_load(1, 1)
    for cur in range(DB):
      step(cur, cur % NI, cur % DB, True)

    @pl.loop(DB, _LOOP_END, step=_U)
    def _(j):
      for t in range(_U):
        cur0 = DB + t
        step(j + t, cur0 % NI, cur0 % DB, True)

    for cur in range(_LOOP_END, C - 2):
      step(cur, cur % NI, cur % DB, True)
    for cur in range(C - 2, C):
      step(cur, cur % NI, cur % DB, False)

    plsc.subcore_barrier()
    _table_io(deg_sh, out_h, sid, cid)

  return deg_kernel


def _make_agg_kernel():
  @functools.partial(
      pl.kernel,
      out_type=jax.ShapeDtypeStruct((NC, N_TAB, D), _f32),
      mesh=_mesh,
      scratch_types=(
          [pltpu.VMEM((K,), jnp.int32) for _ in range(2 * NI)]
          + [pltpu.VMEM((K, D), _f32) for _ in range(DB)]
          + [pltpu.VMEM_SHARED((N_TAB, D), _f32)]
          + [pltpu.SemaphoreType.DMA] * (NI + 2 * DB)
      ),
  )
  def agg_kernel(hp_h, rows_h, cols_h, out_h, *refs):
    ris = refs[0:NI]
    cis = refs[NI:2 * NI]
    bufs = refs[2 * NI:2 * NI + DB]
    agg_sh = refs[2 * NI + DB]
    isems = refs[2 * NI + DB + 1:3 * NI + DB + 1]
    gsems = refs[3 * NI + DB + 1:3 * NI + 2 * DB + 1]
    ssems = refs[3 * NI + 2 * DB + 1:3 * NI + 3 * DB + 1]
    cid = lax.axis_index("c")
    sid = lax.axis_index("s")
    wid = sid * NC + cid

    zeros16 = jnp.zeros((16,), _f32)

    @pl.loop(0, K)
    def _(i):
      for j in range(D // 16):
        bufs[0][i, pl.ds(j * 16, 16)] = zeros16

    _table_io(agg_sh, None, sid, cid, zero_src=bufs[0])
    plsc.subcore_barrier()

    def idx_load(cur, i):
      pltpu.async_copy(rows_h.at[wid, cur], ris[i], isems[i])
      pltpu.async_copy(cols_h.at[wid, cur], cis[i], isems[i])

    def idx_wait(cur, i):
      pltpu.make_async_copy(rows_h.at[wid, cur], ris[i], isems[i]).wait()
      pltpu.make_async_copy(cols_h.at[wid, cur], cis[i], isems[i]).wait()

    def g_start(i, b):
      pltpu.async_copy(hp_h.at[ris[i]], bufs[b], gsems[b])

    def g_wait(i, b):
      pltpu.make_async_copy(hp_h.at[ris[i]], bufs[b], gsems[b]).wait()

    def s_start(i, b):
      pltpu.async_copy(bufs[b], agg_sh.at[cis[i]], ssems[b], add=True)

    def s_wait(i, b):
      pltpu.make_async_copy(bufs[b], agg_sh.at[cis[i]], ssems[b]).wait()

    def step(cur, i, b, load, scat_prev):
      if load:
        idx_load(cur + 2, (i + 2) % NI)
      idx_wait(cur, i)
      g_start(i, b)
      if scat_prev:
        pi, pb = (i - 1) % NI, (b - 1) % DB
        g_wait(pi, pb)
        s_start(pi, pb)
        s_wait(pi, pb)

    idx_load(0, 0)
    idx_load(1, 1)
    step(0, 0, 0, True, False)
    for cur in range(1, DB):
      step(cur, cur % NI, cur % DB, True, True)

    @pl.loop(DB, _LOOP_END, step=_U)
    def _(j):
      for t in range(_U):
        cur0 = DB + t
        step(j + t, cur0 % NI, cur0 % DB, True, True)

    for cur in range(_LOOP_END, C - 2):
      step(cur, cur % NI, cur % DB, True, True)
    for cur in range(C - 2, C):
      step(cur, cur % NI, cur % DB, False, True)
    g_wait((C - 1) % NI, (C - 1) % DB)
    s_start((C - 1) % NI, (C - 1) % DB)
    s_wait((C - 1) % NI, (C - 1) % DB)

    plsc.subcore_barrier()
    _table_io(agg_sh, out_h, sid, cid)

  return agg_kernel


_BLK = 2000


def _g_of(deg_ref):
  d = deg_ref[0, :, 0:1] + deg_ref[1, :, 0:1] + 1.0
  return lax.rsqrt(d)


def _ka_body(x_ref, w_ref, b_ref, deg_ref, o_ref):
  g = _g_of(deg_ref)
  h = jnp.dot(x_ref[...], w_ref[...], preferred_element_type=_f32) + b_ref[...]
  o_ref[...] = g * h


def _kb_body(agg_ref, hp_ref, deg_ref, w_ref, b_ref, o_ref):
  g = _g_of(deg_ref)
  x2 = jnp.maximum(g * (agg_ref[0] + agg_ref[1] + hp_ref[...]), 0.0)
  h = jnp.dot(x2, w_ref[...], preferred_element_type=_f32) + b_ref[...]
  o_ref[...] = g * h


def _kc_body(agg_ref, hp_ref, deg_ref, w_ref, b_ref, o_ref):
  g = _g_of(deg_ref)
  x3 = jnp.maximum(g * (agg_ref[0] + agg_ref[1] + hp_ref[...]), 0.0)
  o_ref[...] = (
      jnp.dot(x3, w_ref[...], preferred_element_type=_f32) + b_ref[...]
  )


def _tc_call(body, stacked_first):
  grid = (N // _BLK,)
  stacked_spec16 = pl.BlockSpec((NC, _BLK, 16), lambda i: (0, i, 0))
  stacked_specD = pl.BlockSpec((NC, _BLK, D), lambda i: (0, i, 0))
  rows_spec = pl.BlockSpec((_BLK, D), lambda i: (i, 0))
  w_spec = pl.BlockSpec((D, D), lambda i: (0, 0))
  b_spec = pl.BlockSpec((1, D), lambda i: (0, 0))
  if not stacked_first:
    in_specs = [rows_spec, w_spec, b_spec, stacked_spec16]
  else:
    in_specs = [stacked_specD, rows_spec, stacked_spec16, w_spec, b_spec]
  return pl.pallas_call(
      body,
      grid=grid,
      in_specs=in_specs,
      out_specs=rows_spec,
      out_shape=jax.ShapeDtypeStruct((N, D), _f32),
  )


_deg_kernel = _make_deg_kernel()
_agg_kernel = _make_agg_kernel()
_ka = _tc_call(_ka_body, False)
_kb = _tc_call(_kb_body, True)
_kc = _tc_call(_kc_body, True)


@jax.jit
def kernel(x, edge_index, W1, b1, W2, b2, Wh, bh):
  pad = E_PAD - E
  pad_i = jnp.arange(pad, dtype=jnp.int32)
  rows_gather = jnp.concatenate([edge_index[0], pad_i % N]).reshape(NW, C, K)
  dummy = N + (pad_i % (N_TAB - N))
  rows_deg = jnp.concatenate([edge_index[0], dummy]).reshape(NW, C, K)
  cols_scatter = jnp.concatenate([edge_index[1], dummy]).reshape(NW, C, K)

  deg = _deg_kernel(rows_deg)
  h1p = _ka(x, W1, b1.reshape(1, D), deg)
  agg1 = _agg_kernel(h1p, rows_gather, cols_scatter)
  h2p = _kb(agg1, h1p, deg, W2, b2.reshape(1, D))
  agg2 = _agg_kernel(h2p, rows_gather, cols_scatter)
  return _kc(agg2, h2p, deg, Wh, bh.reshape(1, D))


# speedup vs baseline: 34.3218x; 1.0342x over previous
import functools

import jax
import jax.numpy as jnp
from jax import lax
from jax.experimental import pallas as pl
from jax.experimental.pallas import tpu as pltpu
from jax.experimental.pallas import tpu_sc as plsc

N = 10000
E = 320000
D = 128

NC = 2
NS = 16
NW = NC * NS

K = 128
C = 80
EPW = C * K
E_PAD = NW * EPW
N_TAB = 10016
RPT = 624
RPT_LAST = N_TAB - 15 * RPT
DB = 3
NI = 5

import math
_U = math.lcm(DB, NI)
_LOOP_END = DB + ((C - 2 - DB) // _U) * _U

_f32 = jnp.float32
_mesh = plsc.VectorSubcoreMesh(
    core_axis_name="c", subcore_axis_name="s", num_cores=NC, num_subcores=NS
)


def _table_io(sh, out_or_none, sid, cid, zero_src=None):
  base = sid * RPT

  def copies(n_rows):
    for t in range(n_rows // K):
      if zero_src is not None:
        pltpu.sync_copy(zero_src, sh.at[pl.ds(base + t * K, K)])
      else:
        pltpu.sync_copy(sh.at[pl.ds(base + t * K, K)],
                        out_or_none.at[cid, pl.ds(base + t * K, K)])
    rem, off = n_rows % K, (n_rows // K) * K
    if rem:
      if zero_src is not None:
        pltpu.sync_copy(zero_src.at[pl.ds(0, rem)],
                        sh.at[pl.ds(base + off, rem)])
      else:
        pltpu.sync_copy(sh.at[pl.ds(base + off, rem)],
                        out_or_none.at[cid, pl.ds(base + off, rem)])

  if RPT_LAST == RPT:
    copies(RPT)
  else:
    @pl.when(sid < NS - 1)
    def _():
      copies(RPT)

    @pl.when(sid == NS - 1)
    def _():
      copies(RPT_LAST)


def _make_deg_kernel():
  @functools.partial(
      pl.kernel,
      out_type=jax.ShapeDtypeStruct((NC, N_TAB, 16), _f32),
      mesh=_mesh,
      scratch_types=(
          [pltpu.VMEM((K,), jnp.int32) for _ in range(NI)]
          + [
              pltpu.VMEM((K, 16), _f32),
              pltpu.VMEM((K, 16), _f32),
              pltpu.VMEM_SHARED((N_TAB, 16), _f32),
          ]
          + [pltpu.SemaphoreType.DMA] * (NI + DB)
      ),
  )
  def deg_kernel(rows_h, out_h, *refs):
    ris = refs[0:NI]
    ones_v, zero_v, deg_sh = refs[NI:NI + 3]
    isems = refs[NI + 3:2 * NI + 3]
    ssems = refs[2 * NI + 3:2 * NI + 3 + DB]
    cid = lax.axis_index("c")
    sid = lax.axis_index("s")
    wid = sid * NC + cid

    ones16 = jnp.ones((16,), _f32)
    zeros16 = jnp.zeros((16,), _f32)

    @pl.loop(0, K)
    def _(i):
      ones_v[i, :] = ones16
      zero_v[i, :] = zeros16

    _table_io(deg_sh, None, sid, cid, zero_src=zero_v)
    plsc.subcore_barrier()

    def idx_load(cur, i):
      pltpu.async_copy(rows_h.at[wid, cur], ris[i], isems[i])

    def idx_wait(cur, i):
      pltpu.make_async_copy(rows_h.at[wid, cur], ris[i], isems[i]).wait()

    def s_start(i, b):
      pltpu.async_copy(ones_v, deg_sh.at[ris[i]], ssems[b], add=True)

    def s_wait(i, b):
      pltpu.make_async_copy(ones_v, deg_sh.at[ris[i]], ssems[b]).wait()

    def step(cur, i, b, load, drain):
      if drain:
        s_wait((i + 2) % NI, b)
      if load:
        idx_load(cur + 2, (i + 2) % NI)
      idx_wait(cur, i)
      s_start(i, b)

    idx_load(0, 0)
    idx_load(1, 1)
    for cur in range(DB):
      step(cur, cur % NI, cur % DB, True, False)

    @pl.loop(DB, _LOOP_END, step=_U)
    def _(j):
      for t in range(_U):
        cur0 = DB + t
        step(j + t, cur0 % NI, cur0 % DB, True, True)

    for cur in range(_LOOP_END, C - 2):
      step(cur, cur % NI, cur % DB, True, True)
    for cur in range(C - 2, C):
      step(cur, cur % NI, cur % DB, False, True)
    for cur in range(C - DB, C):
      s_wait(cur % NI, cur % DB)

    plsc.subcore_barrier()
    _table_io(deg_sh, out_h, sid, cid)

  return deg_kernel


def _make_agg_kernel():
  @functools.partial(
      pl.kernel,
      out_type=jax.ShapeDtypeStruct((NC, N_TAB, D), _f32),
      mesh=_mesh,
      scratch_types=(
          [pltpu.VMEM((K,), jnp.int32) for _ in range(2 * NI)]
          + [pltpu.VMEM((K, D), _f32) for _ in range(DB)]
          + [pltpu.VMEM_SHARED((N_TAB, D), _f32)]
          + [pltpu.SemaphoreType.DMA] * (NI + 2 * DB)
      ),
  )
  def agg_kernel(hp_h, rows_h, cols_h, out_h, *refs):
    ris = refs[0:NI]
    cis = refs[NI:2 * NI]
    bufs = refs[2 * NI:2 * NI + DB]
    agg_sh = refs[2 * NI + DB]
    isems = refs[2 * NI + DB + 1:3 * NI + DB + 1]
    gsems = refs[3 * NI + DB + 1:3 * NI + 2 * DB + 1]
    ssems = refs[3 * NI + 2 * DB + 1:3 * NI + 3 * DB + 1]
    cid = lax.axis_index("c")
    sid = lax.axis_index("s")
    wid = sid * NC + cid

    zeros16 = jnp.zeros((16,), _f32)

    @pl.loop(0, K)
    def _(i):
      for j in range(D // 16):
        bufs[0][i, pl.ds(j * 16, 16)] = zeros16

    _table_io(agg_sh, None, sid, cid, zero_src=bufs[0])
    plsc.subcore_barrier()

    def idx_load(cur, i):
      pltpu.async_copy(rows_h.at[wid, cur], ris[i], isems[i])
      pltpu.async_copy(cols_h.at[wid, cur], cis[i], isems[i])

    def idx_wait(cur, i):
      pltpu.make_async_copy(rows_h.at[wid, cur], ris[i], isems[i]).wait()
      pltpu.make_async_copy(cols_h.at[wid, cur], cis[i], isems[i]).wait()

    def g_start(i, b):
      pltpu.async_copy(hp_h.at[ris[i]], bufs[b], gsems[b])

    def g_wait(i, b):
      pltpu.make_async_copy(hp_h.at[ris[i]], bufs[b], gsems[b]).wait()

    def s_start(i, b):
      pltpu.async_copy(bufs[b], agg_sh.at[cis[i]], ssems[b], add=True)

    def s_wait(i, b):
      pltpu.make_async_copy(bufs[b], agg_sh.at[cis[i]], ssems[b]).wait()

    def step(cur, i, b, load, drain, scat_prev):
      if drain:
        s_wait((i + 2) % NI, b)
      if load:
        idx_load(cur + 2, (i + 2) % NI)
      idx_wait(cur, i)
      g_start(i, b)
      if scat_prev:
        pi, pb = (i - 1) % NI, (b - 1) % DB
        g_wait(pi, pb)
        s_start(pi, pb)

    idx_load(0, 0)
    idx_load(1, 1)
    step(0, 0, 0, True, False, False)
    for cur in range(1, DB):
      step(cur, cur % NI, cur % DB, True, False, True)

    @pl.loop(DB, _LOOP_END, step=_U)
    def _(j):
      for t in range(_U):
        cur0 = DB + t
        step(j + t, cur0 % NI, cur0 % DB, True, True, True)

    for cur in range(_LOOP_END, C - 2):
      step(cur, cur % NI, cur % DB, True, True, True)
    for cur in range(C - 2, C):
      step(cur, cur % NI, cur % DB, False, True, True)
    g_wait((C - 1) % NI, (C - 1) % DB)
    s_start((C - 1) % NI, (C - 1) % DB)
    for cur in range(C - DB, C):
      s_wait(cur % NI, cur % DB)

    plsc.subcore_barrier()
    _table_io(agg_sh, out_h, sid, cid)

  return agg_kernel


_BLK = 2000


def _g_of(deg_ref):
  d = deg_ref[0, :, 0:1] + deg_ref[1, :, 0:1] + 1.0
  return lax.rsqrt(d)


def _ka_body(x_ref, w_ref, b_ref, deg_ref, o_ref):
  g = _g_of(deg_ref)
  h = jnp.dot(x_ref[...], w_ref[...], preferred_element_type=_f32) + b_ref[...]
  o_ref[...] = g * h


def _kb_body(agg_ref, hp_ref, deg_ref, w_ref, b_ref, o_ref):
  g = _g_of(deg_ref)
  x2 = jnp.maximum(g * (agg_ref[0] + agg_ref[1] + hp_ref[...]), 0.0)
  h = jnp.dot(x2, w_ref[...], preferred_element_type=_f32) + b_ref[...]
  o_ref[...] = g * h


def _kc_body(agg_ref, hp_ref, deg_ref, w_ref, b_ref, o_ref):
  g = _g_of(deg_ref)
  x3 = jnp.maximum(g * (agg_ref[0] + agg_ref[1] + hp_ref[...]), 0.0)
  o_ref[...] = (
      jnp.dot(x3, w_ref[...], preferred_element_type=_f32) + b_ref[...]
  )


def _tc_call(body, stacked_first):
  grid = (N // _BLK,)
  stacked_spec16 = pl.BlockSpec((NC, _BLK, 16), lambda i: (0, i, 0))
  stacked_specD = pl.BlockSpec((NC, _BLK, D), lambda i: (0, i, 0))
  rows_spec = pl.BlockSpec((_BLK, D), lambda i: (i, 0))
  w_spec = pl.BlockSpec((D, D), lambda i: (0, 0))
  b_spec = pl.BlockSpec((1, D), lambda i: (0, 0))
  if not stacked_first:
    in_specs = [rows_spec, w_spec, b_spec, stacked_spec16]
  else:
    in_specs = [stacked_specD, rows_spec, stacked_spec16, w_spec, b_spec]
  return pl.pallas_call(
      body,
      grid=grid,
      in_specs=in_specs,
      out_specs=rows_spec,
      out_shape=jax.ShapeDtypeStruct((N, D), _f32),
  )


_deg_kernel = _make_deg_kernel()
_agg_kernel = _make_agg_kernel()
_ka = _tc_call(_ka_body, False)
_kb = _tc_call(_kb_body, True)
_kc = _tc_call(_kc_body, True)


@jax.jit
def kernel(x, edge_index, W1, b1, W2, b2, Wh, bh):
  pad = E_PAD - E
  pad_i = jnp.arange(pad, dtype=jnp.int32)
  rows_gather = jnp.concatenate([edge_index[0], pad_i % N]).reshape(NW, C, K)
  dummy = N + (pad_i % (N_TAB - N))
  rows_deg = jnp.concatenate([edge_index[0], dummy]).reshape(NW, C, K)
  cols_scatter = jnp.concatenate([edge_index[1], dummy]).reshape(NW, C, K)

  deg = _deg_kernel(rows_deg)
  h1p = _ka(x, W1, b1.reshape(1, D), deg)
  agg1 = _agg_kernel(h1p, rows_gather, cols_scatter)
  h2p = _kb(agg1, h1p, deg, W2, b2.reshape(1, D))
  agg2 = _agg_kernel(h2p, rows_gather, cols_scatter)
  return _kc(agg2, h2p, deg, Wh, bh.reshape(1, D))


# speedup vs baseline: 35.0499x; 1.0212x over previous
import functools

import jax
import jax.numpy as jnp
from jax import lax
from jax.experimental import pallas as pl
from jax.experimental.pallas import tpu as pltpu
from jax.experimental.pallas import tpu_sc as plsc

N = 10000
E = 320000
D = 128

NC = 2
NS = 16
NW = NC * NS

K = 128
EPW = E // NW
C = EPW // K
TAIL = EPW - C * K
N_TAB = N
RPT = 632
RPT_LAST = N_TAB - 15 * RPT
DB = 3
NI = 5

import math
_U = math.lcm(DB, NI)
_LOOP_END = DB + ((C - 2 - DB) // _U) * _U

_f32 = jnp.float32
_mesh = plsc.VectorSubcoreMesh(
    core_axis_name="c", subcore_axis_name="s", num_cores=NC, num_subcores=NS
)


def _table_io(sh, out_or_none, sid, cid, zero_src=None):
  base = sid * RPT

  def copies(n_rows):
    for t in range(n_rows // K):
      if zero_src is not None:
        pltpu.sync_copy(zero_src, sh.at[pl.ds(base + t * K, K)])
      else:
        pltpu.sync_copy(sh.at[pl.ds(base + t * K, K)],
                        out_or_none.at[cid, pl.ds(base + t * K, K)])
    rem, off = n_rows % K, (n_rows // K) * K
    if rem:
      if zero_src is not None:
        pltpu.sync_copy(zero_src.at[pl.ds(0, rem)],
                        sh.at[pl.ds(base + off, rem)])
      else:
        pltpu.sync_copy(sh.at[pl.ds(base + off, rem)],
                        out_or_none.at[cid, pl.ds(base + off, rem)])

  if RPT_LAST == RPT:
    copies(RPT)
  else:
    @pl.when(sid < NS - 1)
    def _():
      copies(RPT)

    @pl.when(sid == NS - 1)
    def _():
      copies(RPT_LAST)


def _make_deg_kernel():
  @functools.partial(
      pl.kernel,
      out_type=jax.ShapeDtypeStruct((NC, N_TAB, 16), _f32),
      mesh=_mesh,
      scratch_types=(
          [pltpu.VMEM((K,), jnp.int32) for _ in range(NI)]
          + [
              pltpu.VMEM((TAIL,), jnp.int32),
              pltpu.VMEM((K, 16), _f32),
              pltpu.VMEM((K, 16), _f32),
              pltpu.VMEM_SHARED((N_TAB, 16), _f32),
          ]
          + [pltpu.SemaphoreType.DMA] * (NI + 1 + DB)
      ),
  )
  def deg_kernel(rows_h, out_h, *refs):
    ris = refs[0:NI]
    ri_t, ones_v, zero_v, deg_sh = refs[NI:NI + 4]
    isems = refs[NI + 4:2 * NI + 4]
    tsem = refs[2 * NI + 4]
    ssems = refs[2 * NI + 5:2 * NI + 5 + DB]
    cid = lax.axis_index("c")
    sid = lax.axis_index("s")
    wid = sid * NC + cid
    ebase = wid * EPW

    ones16 = jnp.ones((16,), _f32)
    zeros16 = jnp.zeros((16,), _f32)

    @pl.loop(0, K)
    def _(i):
      ones_v[i, :] = ones16
      zero_v[i, :] = zeros16

    _table_io(deg_sh, None, sid, cid, zero_src=zero_v)
    plsc.subcore_barrier()

    def idx_load(cur, i):
      pltpu.async_copy(rows_h.at[pl.ds(ebase + cur * K, K)], ris[i], isems[i])

    def idx_wait(cur, i):
      pltpu.make_async_copy(
          rows_h.at[pl.ds(ebase + cur * K, K)], ris[i], isems[i]).wait()

    def s_start(i, b):
      pltpu.async_copy(ones_v, deg_sh.at[ris[i]], ssems[b], add=True)

    def s_wait(i, b):
      pltpu.make_async_copy(ones_v, deg_sh.at[ris[i]], ssems[b]).wait()

    def step(cur, i, b, load, drain):
      if drain:
        s_wait((i + 2) % NI, b)
      if load:
        idx_load(cur + 2, (i + 2) % NI)
      idx_wait(cur, i)
      s_start(i, b)

    idx_load(0, 0)
    idx_load(1, 1)
    pltpu.async_copy(rows_h.at[pl.ds(ebase + C * K, TAIL)], ri_t, tsem)
    for cur in range(DB):
      step(cur, cur % NI, cur % DB, True, False)

    @pl.loop(DB, _LOOP_END, step=_U)
    def _(j):
      for t in range(_U):
        cur0 = DB + t
        step(j + t, cur0 % NI, cur0 % DB, True, True)

    for cur in range(_LOOP_END, C - 2):
      step(cur, cur % NI, cur % DB, True, True)
    for cur in range(C - 2, C):
      step(cur, cur % NI, cur % DB, False, True)
    pltpu.make_async_copy(
        rows_h.at[pl.ds(ebase + C * K, TAIL)], ri_t, tsem).wait()
    s_wait((C - DB) % NI, (C - DB) % DB)
    pltpu.async_copy(ones_v.at[pl.ds(0, TAIL)], deg_sh.at[ri_t],
                     ssems[(C - DB) % DB], add=True)
    pltpu.make_async_copy(ones_v.at[pl.ds(0, TAIL)], deg_sh.at[ri_t],
                          ssems[(C - DB) % DB]).wait()
    for cur in range(C - DB + 1, C):
      s_wait(cur % NI, cur % DB)

    plsc.subcore_barrier()
    _table_io(deg_sh, out_h, sid, cid)

  return deg_kernel


def _make_agg_kernel():
  @functools.partial(
      pl.kernel,
      out_type=jax.ShapeDtypeStruct((NC, N_TAB, D), _f32),
      mesh=_mesh,
      scratch_types=(
          [pltpu.VMEM((K,), jnp.int32) for _ in range(2 * NI)]
          + [pltpu.VMEM((TAIL,), jnp.int32) for _ in range(2)]
          + [pltpu.VMEM((K, D), _f32) for _ in range(DB)]
          + [pltpu.VMEM_SHARED((N_TAB, D), _f32)]
          + [pltpu.SemaphoreType.DMA] * (NI + 1 + 2 * DB)
      ),
  )
  def agg_kernel(hp_h, rows_h, cols_h, out_h, *refs):
    ris = refs[0:NI]
    cis = refs[NI:2 * NI]
    ri_t, ci_t = refs[2 * NI:2 * NI + 2]
    bufs = refs[2 * NI + 2:2 * NI + 2 + DB]
    agg_sh = refs[2 * NI + 2 + DB]
    isems = refs[2 * NI + 3 + DB:3 * NI + 3 + DB]
    tsem = refs[3 * NI + 3 + DB]
    gsems = refs[3 * NI + 4 + DB:3 * NI + 4 + 2 * DB]
    ssems = refs[3 * NI + 4 + 2 * DB:3 * NI + 4 + 3 * DB]
    cid = lax.axis_index("c")
    sid = lax.axis_index("s")
    wid = sid * NC + cid
    ebase = wid * EPW

    zeros16 = jnp.zeros((16,), _f32)

    @pl.loop(0, K)
    def _(i):
      for j in range(D // 16):
        bufs[0][i, pl.ds(j * 16, 16)] = zeros16

    _table_io(agg_sh, None, sid, cid, zero_src=bufs[0])
    plsc.subcore_barrier()

    def idx_load(cur, i):
      sl = pl.ds(ebase + cur * K, K)
      pltpu.async_copy(rows_h.at[sl], ris[i], isems[i])
      pltpu.async_copy(cols_h.at[sl], cis[i], isems[i])

    def idx_wait(cur, i):
      sl = pl.ds(ebase + cur * K, K)
      pltpu.make_async_copy(rows_h.at[sl], ris[i], isems[i]).wait()
      pltpu.make_async_copy(cols_h.at[sl], cis[i], isems[i]).wait()

    def g_start(i, b):
      pltpu.async_copy(hp_h.at[ris[i]], bufs[b], gsems[b])

    def g_wait(i, b):
      pltpu.make_async_copy(hp_h.at[ris[i]], bufs[b], gsems[b]).wait()

    def s_start(i, b):
      pltpu.async_copy(bufs[b], agg_sh.at[cis[i]], ssems[b], add=True)

    def s_wait(i, b):
      pltpu.make_async_copy(bufs[b], agg_sh.at[cis[i]], ssems[b]).wait()

    def step(cur, i, b, load, drain, scat_prev):
      if drain:
        s_wait((i + 2) % NI, b)
      if load:
        idx_load(cur + 2, (i + 2) % NI)
      idx_wait(cur, i)
      g_start(i, b)
      if scat_prev:
        pi, pb = (i - 1) % NI, (b - 1) % DB
        g_wait(pi, pb)
        s_start(pi, pb)

    idx_load(0, 0)
    idx_load(1, 1)
    tsl = pl.ds(ebase + C * K, TAIL)
    pltpu.async_copy(rows_h.at[tsl], ri_t, tsem)
    pltpu.async_copy(cols_h.at[tsl], ci_t, tsem)
    step(0, 0, 0, True, False, False)
    for cur in range(1, DB):
      step(cur, cur % NI, cur % DB, True, False, True)

    @pl.loop(DB, _LOOP_END, step=_U)
    def _(j):
      for t in range(_U):
        cur0 = DB + t
        step(j + t, cur0 % NI, cur0 % DB, True, True, True)

    for cur in range(_LOOP_END, C - 2):
      step(cur, cur % NI, cur % DB, True, True, True)
    for cur in range(C - 2, C):
      step(cur, cur % NI, cur % DB, False, True, True)
    g_wait((C - 1) % NI, (C - 1) % DB)
    s_start((C - 1) % NI, (C - 1) % DB)
    pltpu.make_async_copy(rows_h.at[tsl], ri_t, tsem).wait()
    pltpu.make_async_copy(cols_h.at[tsl], ci_t, tsem).wait()
    tb = (C - DB) % DB
    s_wait((C - DB) % NI, tb)
    tbuf = bufs[tb].at[pl.ds(0, TAIL)]
    pltpu.async_copy(hp_h.at[ri_t], tbuf, gsems[tb])
    pltpu.make_async_copy(hp_h.at[ri_t], tbuf, gsems[tb]).wait()
    pltpu.async_copy(tbuf, agg_sh.at[ci_t], ssems[tb], add=True)
    pltpu.make_async_copy(tbuf, agg_sh.at[ci_t], ssems[tb]).wait()
    for cur in range(C - DB + 1, C):
      s_wait(cur % NI, cur % DB)

    plsc.subcore_barrier()
    _table_io(agg_sh, out_h, sid, cid)

  return agg_kernel


_BLK = 2000


def _g_of(deg_ref):
  d = deg_ref[0, :, 0:1] + deg_ref[1, :, 0:1] + 1.0
  return lax.rsqrt(d)


def _ka_body(x_ref, w_ref, b_ref, deg_ref, o_ref):
  g = _g_of(deg_ref)
  h = jnp.dot(x_ref[...], w_ref[...], preferred_element_type=_f32) + b_ref[...]
  o_ref[...] = g * h


def _kb_body(agg_ref, hp_ref, deg_ref, w_ref, b_ref, o_ref):
  g = _g_of(deg_ref)
  x2 = jnp.maximum(g * (agg_ref[0] + agg_ref[1] + hp_ref[...]), 0.0)
  h = jnp.dot(x2, w_ref[...], preferred_element_type=_f32) + b_ref[...]
  o_ref[...] = g * h


def _kc_body(agg_ref, hp_ref, deg_ref, w_ref, b_ref, o_ref):
  g = _g_of(deg_ref)
  x3 = jnp.maximum(g * (agg_ref[0] + agg_ref[1] + hp_ref[...]), 0.0)
  o_ref[...] = (
      jnp.dot(x3, w_ref[...], preferred_element_type=_f32) + b_ref[...]
  )


def _tc_call(body, stacked_first):
  grid = (N // _BLK,)
  stacked_spec16 = pl.BlockSpec((NC, _BLK, 16), lambda i: (0, i, 0))
  stacked_specD = pl.BlockSpec((NC, _BLK, D), lambda i: (0, i, 0))
  rows_spec = pl.BlockSpec((_BLK, D), lambda i: (i, 0))
  w_spec = pl.BlockSpec((D, D), lambda i: (0, 0))
  b_spec = pl.BlockSpec((1, D), lambda i: (0, 0))
  if not stacked_first:
    in_specs = [rows_spec, w_spec, b_spec, stacked_spec16]
  else:
    in_specs = [stacked_specD, rows_spec, stacked_spec16, w_spec, b_spec]
  return pl.pallas_call(
      body,
      grid=grid,
      in_specs=in_specs,
      out_specs=rows_spec,
      out_shape=jax.ShapeDtypeStruct((N, D), _f32),
  )


_deg_kernel = _make_deg_kernel()
_agg_kernel = _make_agg_kernel()
_ka = _tc_call(_ka_body, False)
_kb = _tc_call(_kb_body, True)
_kc = _tc_call(_kc_body, True)


@jax.jit
def kernel(x, edge_index, W1, b1, W2, b2, Wh, bh):
  rows = edge_index[0]
  cols = edge_index[1]

  deg = _deg_kernel(rows)
  h1p = _ka(x, W1, b1.reshape(1, D), deg)
  agg1 = _agg_kernel(h1p, rows, cols)
  h2p = _kb(agg1, h1p, deg, W2, b2.reshape(1, D))
  agg2 = _agg_kernel(h2p, rows, cols)
  return _kc(agg2, h2p, deg, Wh, bh.reshape(1, D))
